# baseline (speedup 1.0000x reference)
import functools

import jax
import jax.numpy as jnp
from jax import lax
from jax.experimental import pallas as pl
from jax.experimental.pallas import tpu as pltpu
from jax.experimental.pallas import tpu_sc as plsc

N_NODES = 10000
N_EDGES = 320000
N_PAD = 10240
NC, NS = 2, 16
NW = NC * NS
CHUNK = 128
CHUNKS_PW = 80
EDGES_PW = CHUNK * CHUNKS_PW
E_PAD = NW * EDGES_PW
ROWS_PW = N_PAD // NS

_MESH = plsc.VectorSubcoreMesh(
    core_axis_name="c", subcore_axis_name="s", num_cores=NC, num_subcores=NS
)


def _worker_ids():
    c = lax.axis_index("c")
    s = lax.axis_index("s")
    return c, s, c * NS + s


def _deg_body(dst_hbm, ones_hbm, zer_hbm, out_hbm, dstv, onesv, acc):
    c, s, w = _worker_ids()
    pltpu.sync_copy(zer_hbm, acc.at[pl.ds(s * ROWS_PW, ROWS_PW)])
    pltpu.sync_copy(dst_hbm.at[pl.ds(w * CHUNKS_PW, CHUNKS_PW)], dstv)
    pltpu.sync_copy(ones_hbm, onesv)
    plsc.subcore_barrier()

    @pl.loop(0, CHUNKS_PW)
    def _(j):
        pltpu.sync_copy(onesv, acc.at[dstv.at[j]], add=True)

    plsc.subcore_barrier()
    pltpu.sync_copy(
        acc.at[pl.ds(s * ROWS_PW, ROWS_PW)],
        out_hbm.at[c, pl.ds(s * ROWS_PW, ROWS_PW)],
    )


_SC_PARAMS = pltpu.CompilerParams(use_tc_tiling_on_sc=False)


DEG_W = 16


def _sc_degree(dst_p, ones, zer):
    return pl.kernel(
        _deg_body,
        out_type=jax.ShapeDtypeStruct((NC, N_PAD, DEG_W), jnp.float32),
        mesh=_MESH,
        compiler_params=_SC_PARAMS,
        scratch_types=[
            pltpu.VMEM((CHUNKS_PW, CHUNK), jnp.int32),
            pltpu.VMEM((CHUNK, DEG_W), jnp.float32),
            pltpu.VMEM_SHARED((N_PAD, DEG_W), jnp.float32),
        ],
    )(dst_p, ones, zer)


PA, PD = 2, 1
P = PA + PD


def _agg_body(*refs):
    H = len(refs) - 11
    tables = refs[:H]
    (src_hbm, dst_hbm, zer_hbm, out_hbm,
     srcv, dstv, rows, table, acc, gsem, ssem) = refs[H:]
    c, s, w = _worker_ids()
    pltpu.sync_copy(src_hbm.at[pl.ds(w * CHUNKS_PW, CHUNKS_PW)], srcv)
    pltpu.sync_copy(dst_hbm.at[pl.ds(w * CHUNKS_PW, CHUNKS_PW)], dstv)

    for h, g_hbm in enumerate(tables):
        pltpu.sync_copy(
            g_hbm.at[pl.ds(s * ROWS_PW, ROWS_PW)],
            table.at[pl.ds(s * ROWS_PW, ROWS_PW)],
        )

        @pl.loop(0, ROWS_PW // CHUNK)
        def _(r):
            pltpu.sync_copy(zer_hbm, acc.at[pl.ds(s * ROWS_PW + r * CHUNK, CHUNK)])

        plsc.subcore_barrier()

        for q in range(PA):
            pltpu.async_copy(table.at[srcv.at[q]], rows.at[q], gsem.at[q])

        @pl.loop(0, CHUNKS_PW)
        def _(j):
            p = lax.rem(j, P)
            pn = lax.rem(j + PA, P)

            @pl.when(j >= PD)
            def _():
                pltpu.make_async_copy(
                    rows.at[pn], acc.at[pl.ds(0, CHUNK)], ssem.at[pn]
                ).wait()

            @pl.when(j + PA < CHUNKS_PW)
            def _():
                pltpu.async_copy(table.at[srcv.at[j + PA]], rows.at[pn], gsem.at[pn])

            pltpu.make_async_copy(
                table.at[pl.ds(0, CHUNK)], rows.at[p], gsem.at[p]
            ).wait()
            pltpu.async_copy(rows.at[p], acc.at[dstv.at[j]], ssem.at[p], add=True)

        for jt in range(CHUNKS_PW - PD, CHUNKS_PW):
            pltpu.make_async_copy(
                rows.at[jt % P], acc.at[pl.ds(0, CHUNK)], ssem.at[jt % P]
            ).wait()
        plsc.subcore_barrier()
        pltpu.sync_copy(
            acc.at[pl.ds(s * ROWS_PW, ROWS_PW)],
            out_hbm.at[h, c, pl.ds(s * ROWS_PW, ROWS_PW)],
        )


D_AGG = 64


def _sc_aggregate(tables, src_p, dst_p, zer):
    H = len(tables)
    return pl.kernel(
        _agg_body,
        out_type=jax.ShapeDtypeStruct((H, NC, N_PAD, D_AGG), jnp.float32),
        mesh=_MESH,
        compiler_params=_SC_PARAMS,
        scratch_types=[
            pltpu.VMEM((CHUNKS_PW, CHUNK), jnp.int32),
            pltpu.VMEM((CHUNKS_PW, CHUNK), jnp.int32),
            pltpu.VMEM((P, CHUNK, D_AGG), jnp.float32),
            pltpu.VMEM_SHARED((N_PAD, D_AGG), jnp.float32),
            pltpu.VMEM_SHARED((N_PAD, D_AGG), jnp.float32),
            pltpu.SemaphoreType.DMA((P,)),
            pltpu.SemaphoreType.DMA((P,)),
        ],
    )(*tables, src_p, dst_p, zer)


_BR = 1024


def _mm_in_body(x_ref, wl_ref, wr_ref, d0_ref, d1_ref, ol_ref, or_ref):
    dinv = lax.rsqrt(d0_ref[:, :1] + d1_ref[:, :1] + 1.0)
    ol_ref[...] = (
        jnp.dot(x_ref[...], wl_ref[...], preferred_element_type=jnp.float32) * dinv
    )
    or_ref[...] = (
        jnp.dot(x_ref[...], wr_ref[...], preferred_element_type=jnp.float32) * dinv
    )


def _tc_in(x, wl, wr, d0, d1):
    k = x.shape[1]
    half = jax.ShapeDtypeStruct((N_PAD, D_AGG), jnp.float32)
    return pl.pallas_call(
        _mm_in_body,
        grid=(N_PAD // _BR,),
        in_specs=[
            pl.BlockSpec((_BR, k), lambda i: (i, 0)),
            pl.BlockSpec((k, D_AGG), lambda i: (0, 0)),
            pl.BlockSpec((k, D_AGG), lambda i: (0, 0)),
            pl.BlockSpec((_BR, DEG_W), lambda i: (i, 0)),
            pl.BlockSpec((_BR, DEG_W), lambda i: (i, 0)),
        ],
        out_specs=[
            pl.BlockSpec((_BR, D_AGG), lambda i: (i, 0)),
            pl.BlockSpec((_BR, D_AGG), lambda i: (i, 0)),
        ],
        out_shape=[half, half],
    )(x, wl, wr, d0, d1)


def _mid_body(a0l, a1l, gl, a0r, a1r, gr, d0, d1, bl, br, wt, wb, o_ref):
    dinv = lax.rsqrt(d0[:, :1] + d1[:, :1] + 1.0)
    hl = jnp.maximum(dinv * (a0l[...] + a1l[...] + gl[...]) + bl[...], 0.0)
    hr = jnp.maximum(dinv * (a0r[...] + a1r[...] + gr[...]) + br[...], 0.0)
    o_ref[...] = (
        jnp.dot(hl, wt[...], preferred_element_type=jnp.float32)
        + jnp.dot(hr, wb[...], preferred_element_type=jnp.float32)
    ) * dinv


def _tc_mid(a0l, a1l, gl, a0r, a1r, gr, d0, d1, bl, br, wt, wb, dout):
    half = pl.BlockSpec((_BR, D_AGG), lambda i: (i, 0))
    degs = pl.BlockSpec((_BR, DEG_W), lambda i: (i, 0))
    return pl.pallas_call(
        _mid_body,
        grid=(N_PAD // _BR,),
        in_specs=[
            half, half, half, half, half, half, degs, degs,
            pl.BlockSpec((1, D_AGG), lambda i: (0, 0)),
            pl.BlockSpec((1, D_AGG), lambda i: (0, 0)),
            pl.BlockSpec((D_AGG, dout), lambda i: (0, 0)),
            pl.BlockSpec((D_AGG, dout), lambda i: (0, 0)),
        ],
        out_specs=pl.BlockSpec((_BR, dout), lambda i: (i, 0)),
        out_shape=jax.ShapeDtypeStruct((N_PAD, dout), jnp.float32),
    )(a0l, a1l, gl, a0r, a1r, gr, d0, d1, bl, br, wt, wb)


def _out_body(a0_ref, a1_ref, g_ref, d0_ref, d1_ref, b_ref, w_ref, bc_ref, o_ref):
    dinv = lax.rsqrt(d0_ref[:, :1] + d1_ref[:, :1] + 1.0)
    h = jnp.maximum(
        dinv * (a0_ref[...] + a1_ref[...] + g_ref[...]) + b_ref[...], 0.0
    )
    o_ref[...] = (
        jnp.dot(h, w_ref[...], preferred_element_type=jnp.float32) + bc_ref[...]
    )


def _tc_out(a0, a1, g, d0, d1, b, w, bc, dout):
    k = g.shape[1]
    return pl.pallas_call(
        _out_body,
        grid=(N_PAD // _BR,),
        in_specs=[
            pl.BlockSpec((_BR, k), lambda i: (i, 0)),
            pl.BlockSpec((_BR, k), lambda i: (i, 0)),
            pl.BlockSpec((_BR, k), lambda i: (i, 0)),
            pl.BlockSpec((_BR, DEG_W), lambda i: (i, 0)),
            pl.BlockSpec((_BR, DEG_W), lambda i: (i, 0)),
            pl.BlockSpec((1, k), lambda i: (0, 0)),
            pl.BlockSpec((k, dout), lambda i: (0, 0)),
            pl.BlockSpec((1, dout), lambda i: (0, 0)),
        ],
        out_specs=pl.BlockSpec((_BR, dout), lambda i: (i, 0)),
        out_shape=jax.ShapeDtypeStruct((N_PAD, dout), jnp.float32),
    )(a0, a1, g, d0, d1, b, w, bc)


def kernel(X, edge_index, W1, b1, W2, b2, Wc, bc):
    f32 = jnp.float32
    src = edge_index[0].astype(jnp.int32)
    dst = edge_index[1].astype(jnp.int32)
    npad = E_PAD - N_EDGES
    pad_dst = N_NODES + (jnp.arange(npad, dtype=jnp.int32) % (N_PAD - N_NODES))
    src_p = jnp.concatenate([src, jnp.zeros((npad,), jnp.int32)]).reshape(
        E_PAD // CHUNK, CHUNK
    )
    dst_p = jnp.concatenate([dst, pad_dst]).reshape(E_PAD // CHUNK, CHUNK)

    Xp = jnp.concatenate([X, jnp.zeros((N_PAD - N_NODES, X.shape[1]), f32)])
    ones_c = jnp.ones((CHUNK, DEG_W), f32)
    zer_r1 = jnp.zeros((ROWS_PW, DEG_W), f32)
    zer64 = jnp.zeros((CHUNK, 64), f32)

    degs = _sc_degree(dst_p, ones_c, zer_r1)
    d0, d1 = degs[0], degs[1]

    g1l, g1r = _tc_in(Xp, W1[:, :D_AGG], W1[:, D_AGG:], d0, d1)
    a1 = _sc_aggregate([g1l, g1r], src_p, dst_p, zer64)
    g2 = _tc_mid(
        a1[0, 0], a1[0, 1], g1l, a1[1, 0], a1[1, 1], g1r, d0, d1,
        b1[:D_AGG].reshape(1, -1), b1[D_AGG:].reshape(1, -1),
        W2[:D_AGG], W2[D_AGG:], 64,
    )
    a2 = _sc_aggregate([g2], src_p, dst_p, zer64)
    logits = _tc_out(
        a2[0, 0], a2[0, 1], g2, d0, d1, b2.reshape(1, -1), Wc, bc.reshape(1, -1), 32
    )
    return logits[:N_NODES]


# speedup vs baseline: 1.0732x; 1.0732x over previous
import functools

import jax
import jax.numpy as jnp
from jax import lax
from jax.experimental import pallas as pl
from jax.experimental.pallas import tpu as pltpu
from jax.experimental.pallas import tpu_sc as plsc

N_NODES = 10000
N_EDGES = 320000
N_PAD = 10240
NC, NS = 2, 16
NW = NC * NS
CHUNK = 128
CHUNKS_PW = 80
EDGES_PW = CHUNK * CHUNKS_PW
E_PAD = NW * EDGES_PW
ROWS_PW = N_PAD // NS

_MESH = plsc.VectorSubcoreMesh(
    core_axis_name="c", subcore_axis_name="s", num_cores=NC, num_subcores=NS
)


def _worker_ids():
    c = lax.axis_index("c")
    s = lax.axis_index("s")
    return c, s, c * NS + s


def _deg_body(dst_hbm, ones_hbm, zer_hbm, out_hbm, dstv, onesv, acc):
    c, s, w = _worker_ids()
    pltpu.sync_copy(zer_hbm, acc.at[pl.ds(s * ROWS_PW, ROWS_PW)])
    pltpu.sync_copy(dst_hbm.at[pl.ds(w * CHUNKS_PW, CHUNKS_PW)], dstv)
    pltpu.sync_copy(ones_hbm, onesv)
    plsc.subcore_barrier()

    @pl.loop(0, CHUNKS_PW)
    def _(j):
        pltpu.sync_copy(onesv, acc.at[dstv.at[j]], add=True)

    plsc.subcore_barrier()
    pltpu.sync_copy(
        acc.at[pl.ds(s * ROWS_PW, ROWS_PW)],
        out_hbm.at[c, pl.ds(s * ROWS_PW, ROWS_PW)],
    )


_SC_PARAMS = pltpu.CompilerParams(use_tc_tiling_on_sc=False)


DEG_W = 16


def _sc_degree(dst_p, ones, zer):
    return pl.kernel(
        _deg_body,
        out_type=jax.ShapeDtypeStruct((NC, N_PAD, DEG_W), jnp.float32),
        mesh=_MESH,
        compiler_params=_SC_PARAMS,
        scratch_types=[
            pltpu.VMEM((CHUNKS_PW, CHUNK), jnp.int32),
            pltpu.VMEM((CHUNK, DEG_W), jnp.float32),
            pltpu.VMEM_SHARED((N_PAD, DEG_W), jnp.float32),
        ],
    )(dst_p, ones, zer)


PA, PD = 1, 2
P = PA + PD


def _agg_body(*refs):
    H = len(refs) - 11
    tables = refs[:H]
    (src_hbm, dst_hbm, zer_hbm, out_hbm,
     srcv, dstv, rows, table, acc, gsem, ssem) = refs[H:]
    c, s, w = _worker_ids()
    pltpu.sync_copy(src_hbm.at[pl.ds(w * CHUNKS_PW, CHUNKS_PW)], srcv)
    pltpu.sync_copy(dst_hbm.at[pl.ds(w * CHUNKS_PW, CHUNKS_PW)], dstv)

    for h, g_hbm in enumerate(tables):
        pltpu.sync_copy(
            g_hbm.at[pl.ds(s * ROWS_PW, ROWS_PW)],
            table.at[pl.ds(s * ROWS_PW, ROWS_PW)],
        )

        @pl.loop(0, ROWS_PW // CHUNK)
        def _(r):
            pltpu.sync_copy(zer_hbm, acc.at[pl.ds(s * ROWS_PW + r * CHUNK, CHUNK)])

        plsc.subcore_barrier()

        for q in range(PA):
            pltpu.async_copy(table.at[srcv.at[q]], rows.at[q], gsem.at[q])

        @pl.loop(0, CHUNKS_PW)
        def _(j):
            p = lax.rem(j, P)
            pn = lax.rem(j + PA, P)

            @pl.when(j >= PD)
            def _():
                pltpu.make_async_copy(
                    rows.at[pn], acc.at[pl.ds(0, CHUNK)], ssem.at[pn]
                ).wait()

            @pl.when(j + PA < CHUNKS_PW)
            def _():
                pltpu.async_copy(table.at[srcv.at[j + PA]], rows.at[pn], gsem.at[pn])

            pltpu.make_async_copy(
                table.at[pl.ds(0, CHUNK)], rows.at[p], gsem.at[p]
            ).wait()
            pltpu.async_copy(rows.at[p], acc.at[dstv.at[j]], ssem.at[p], add=True)

        for jt in range(CHUNKS_PW - PD, CHUNKS_PW):
            pltpu.make_async_copy(
                rows.at[jt % P], acc.at[pl.ds(0, CHUNK)], ssem.at[jt % P]
            ).wait()
        plsc.subcore_barrier()
        pltpu.sync_copy(
            acc.at[pl.ds(s * ROWS_PW, ROWS_PW)],
            out_hbm.at[h, c, pl.ds(s * ROWS_PW, ROWS_PW)],
        )


D_AGG = 64


def _sc_aggregate(tables, src_p, dst_p, zer):
    H = len(tables)
    return pl.kernel(
        _agg_body,
        out_type=jax.ShapeDtypeStruct((H, NC, N_PAD, D_AGG), jnp.float32),
        mesh=_MESH,
        compiler_params=_SC_PARAMS,
        scratch_types=[
            pltpu.VMEM((CHUNKS_PW, CHUNK), jnp.int32),
            pltpu.VMEM((CHUNKS_PW, CHUNK), jnp.int32),
            pltpu.VMEM((P, CHUNK, D_AGG), jnp.float32),
            pltpu.VMEM_SHARED((N_PAD, D_AGG), jnp.float32),
            pltpu.VMEM_SHARED((N_PAD, D_AGG), jnp.float32),
            pltpu.SemaphoreType.DMA((P,)),
            pltpu.SemaphoreType.DMA((P,)),
        ],
    )(*tables, src_p, dst_p, zer)


_BR = 1024


def _mm_in_body(x_ref, wl_ref, wr_ref, d0_ref, d1_ref, ol_ref, or_ref):
    dinv = lax.rsqrt(d0_ref[:, :1] + d1_ref[:, :1] + 1.0)
    ol_ref[...] = (
        jnp.dot(x_ref[...], wl_ref[...], preferred_element_type=jnp.float32) * dinv
    )
    or_ref[...] = (
        jnp.dot(x_ref[...], wr_ref[...], preferred_element_type=jnp.float32) * dinv
    )


def _tc_in(x, wl, wr, d0, d1):
    k = x.shape[1]
    half = jax.ShapeDtypeStruct((N_PAD, D_AGG), jnp.float32)
    return pl.pallas_call(
        _mm_in_body,
        grid=(N_PAD // _BR,),
        in_specs=[
            pl.BlockSpec((_BR, k), lambda i: (i, 0)),
            pl.BlockSpec((k, D_AGG), lambda i: (0, 0)),
            pl.BlockSpec((k, D_AGG), lambda i: (0, 0)),
            pl.BlockSpec((_BR, DEG_W), lambda i: (i, 0)),
            pl.BlockSpec((_BR, DEG_W), lambda i: (i, 0)),
        ],
        out_specs=[
            pl.BlockSpec((_BR, D_AGG), lambda i: (i, 0)),
            pl.BlockSpec((_BR, D_AGG), lambda i: (i, 0)),
        ],
        out_shape=[half, half],
    )(x, wl, wr, d0, d1)


def _mid_body(a0l, a1l, gl, a0r, a1r, gr, d0, d1, bl, br, wt, wb, o_ref):
    dinv = lax.rsqrt(d0[:, :1] + d1[:, :1] + 1.0)
    hl = jnp.maximum(dinv * (a0l[...] + a1l[...] + gl[...]) + bl[...], 0.0)
    hr = jnp.maximum(dinv * (a0r[...] + a1r[...] + gr[...]) + br[...], 0.0)
    o_ref[...] = (
        jnp.dot(hl, wt[...], preferred_element_type=jnp.float32)
        + jnp.dot(hr, wb[...], preferred_element_type=jnp.float32)
    ) * dinv


def _tc_mid(a0l, a1l, gl, a0r, a1r, gr, d0, d1, bl, br, wt, wb, dout):
    half = pl.BlockSpec((_BR, D_AGG), lambda i: (i, 0))
    degs = pl.BlockSpec((_BR, DEG_W), lambda i: (i, 0))
    return pl.pallas_call(
        _mid_body,
        grid=(N_PAD // _BR,),
        in_specs=[
            half, half, half, half, half, half, degs, degs,
            pl.BlockSpec((1, D_AGG), lambda i: (0, 0)),
            pl.BlockSpec((1, D_AGG), lambda i: (0, 0)),
            pl.BlockSpec((D_AGG, dout), lambda i: (0, 0)),
            pl.BlockSpec((D_AGG, dout), lambda i: (0, 0)),
        ],
        out_specs=pl.BlockSpec((_BR, dout), lambda i: (i, 0)),
        out_shape=jax.ShapeDtypeStruct((N_PAD, dout), jnp.float32),
    )(a0l, a1l, gl, a0r, a1r, gr, d0, d1, bl, br, wt, wb)


def _out_body(a0_ref, a1_ref, g_ref, d0_ref, d1_ref, b_ref, w_ref, bc_ref, o_ref):
    dinv = lax.rsqrt(d0_ref[:, :1] + d1_ref[:, :1] + 1.0)
    h = jnp.maximum(
        dinv * (a0_ref[...] + a1_ref[...] + g_ref[...]) + b_ref[...], 0.0
    )
    o_ref[...] = (
        jnp.dot(h, w_ref[...], preferred_element_type=jnp.float32) + bc_ref[...]
    )


def _tc_out(a0, a1, g, d0, d1, b, w, bc, dout):
    k = g.shape[1]
    return pl.pallas_call(
        _out_body,
        grid=(N_PAD // _BR,),
        in_specs=[
            pl.BlockSpec((_BR, k), lambda i: (i, 0)),
            pl.BlockSpec((_BR, k), lambda i: (i, 0)),
            pl.BlockSpec((_BR, k), lambda i: (i, 0)),
            pl.BlockSpec((_BR, DEG_W), lambda i: (i, 0)),
            pl.BlockSpec((_BR, DEG_W), lambda i: (i, 0)),
            pl.BlockSpec((1, k), lambda i: (0, 0)),
            pl.BlockSpec((k, dout), lambda i: (0, 0)),
            pl.BlockSpec((1, dout), lambda i: (0, 0)),
        ],
        out_specs=pl.BlockSpec((_BR, dout), lambda i: (i, 0)),
        out_shape=jax.ShapeDtypeStruct((N_PAD, dout), jnp.float32),
    )(a0, a1, g, d0, d1, b, w, bc)


def kernel(X, edge_index, W1, b1, W2, b2, Wc, bc):
    f32 = jnp.float32
    src = edge_index[0].astype(jnp.int32)
    dst = edge_index[1].astype(jnp.int32)
    npad = E_PAD - N_EDGES
    pad_dst = N_NODES + (jnp.arange(npad, dtype=jnp.int32) % (N_PAD - N_NODES))
    src_p = jnp.concatenate([src, jnp.zeros((npad,), jnp.int32)]).reshape(
        E_PAD // CHUNK, CHUNK
    )
    dst_p = jnp.concatenate([dst, pad_dst]).reshape(E_PAD // CHUNK, CHUNK)

    Xp = jnp.concatenate([X, jnp.zeros((N_PAD - N_NODES, X.shape[1]), f32)])
    ones_c = jnp.ones((CHUNK, DEG_W), f32)
    zer_r1 = jnp.zeros((ROWS_PW, DEG_W), f32)
    zer64 = jnp.zeros((CHUNK, 64), f32)

    degs = _sc_degree(dst_p, ones_c, zer_r1)
    d0, d1 = degs[0], degs[1]

    g1l, g1r = _tc_in(Xp, W1[:, :D_AGG], W1[:, D_AGG:], d0, d1)
    a1 = _sc_aggregate([g1l, g1r], src_p, dst_p, zer64)
    g2 = _tc_mid(
        a1[0, 0], a1[0, 1], g1l, a1[1, 0], a1[1, 1], g1r, d0, d1,
        b1[:D_AGG].reshape(1, -1), b1[D_AGG:].reshape(1, -1),
        W2[:D_AGG], W2[D_AGG:], 64,
    )
    a2 = _sc_aggregate([g2], src_p, dst_p, zer64)
    logits = _tc_out(
        a2[0, 0], a2[0, 1], g2, d0, d1, b2.reshape(1, -1), Wc, bc.reshape(1, -1), 32
    )
    return logits[:N_NODES]


# speedup vs baseline: 1.0735x; 1.0003x over previous
import functools

import jax
import jax.numpy as jnp
from jax import lax
from jax.experimental import pallas as pl
from jax.experimental.pallas import tpu as pltpu
from jax.experimental.pallas import tpu_sc as plsc

N_NODES = 10000
N_EDGES = 320000
N_PAD = 10240
NC, NS = 2, 16
NW = NC * NS
CHUNK = 128
CHUNKS_PW = 80
EDGES_PW = CHUNK * CHUNKS_PW
E_PAD = NW * EDGES_PW
ROWS_PW = N_PAD // NS

def _mesh():
    return plsc.VectorSubcoreMesh(
        core_axis_name="c", subcore_axis_name="s", num_cores=NC, num_subcores=NS
    )


def _worker_ids():
    c = lax.axis_index("c")
    s = lax.axis_index("s")
    return c, s, c * NS + s


def _deg_body(dst_hbm, ones_hbm, zer_hbm, out_hbm, dstv, onesv, acc):
    c, s, w = _worker_ids()
    pltpu.sync_copy(zer_hbm, acc.at[pl.ds(s * ROWS_PW, ROWS_PW)])
    pltpu.sync_copy(dst_hbm.at[pl.ds(w * CHUNKS_PW, CHUNKS_PW)], dstv)
    pltpu.sync_copy(ones_hbm, onesv)
    plsc.subcore_barrier()

    @pl.loop(0, CHUNKS_PW)
    def _(j):
        pltpu.sync_copy(onesv, acc.at[dstv.at[j]], add=True)

    plsc.subcore_barrier()
    pltpu.sync_copy(
        acc.at[pl.ds(s * ROWS_PW, ROWS_PW)],
        out_hbm.at[c, pl.ds(s * ROWS_PW, ROWS_PW)],
    )


_SC_PARAMS = pltpu.CompilerParams(use_tc_tiling_on_sc=False)


DEG_W = 16


def _sc_degree(dst_p, ones, zer):
    return pl.kernel(
        _deg_body,
        out_type=jax.ShapeDtypeStruct((NC, N_PAD, DEG_W), jnp.float32),
        mesh=_mesh(),
        compiler_params=_SC_PARAMS,
        scratch_types=[
            pltpu.VMEM((CHUNKS_PW, CHUNK), jnp.int32),
            pltpu.VMEM((CHUNK, DEG_W), jnp.float32),
            pltpu.VMEM_SHARED((N_PAD, DEG_W), jnp.float32),
        ],
    )(dst_p, ones, zer)


PA, PD = 1, 2
P = PA + PD


def _agg_body(*refs):
    H = len(refs) - 11
    tables = refs[:H]
    (src_hbm, dst_hbm, zer_hbm, out_hbm,
     srcv, dstv, rows, table, acc, gsem, ssem) = refs[H:]
    c, s, w = _worker_ids()
    pltpu.sync_copy(src_hbm.at[pl.ds(w * CHUNKS_PW, CHUNKS_PW)], srcv)
    pltpu.sync_copy(dst_hbm.at[pl.ds(w * CHUNKS_PW, CHUNKS_PW)], dstv)

    for h, g_hbm in enumerate(tables):
        pltpu.sync_copy(
            g_hbm.at[pl.ds(s * ROWS_PW, ROWS_PW)],
            table.at[pl.ds(s * ROWS_PW, ROWS_PW)],
        )

        @pl.loop(0, ROWS_PW // CHUNK)
        def _(r):
            pltpu.sync_copy(zer_hbm, acc.at[pl.ds(s * ROWS_PW + r * CHUNK, CHUNK)])

        plsc.subcore_barrier()

        for q in range(PA):
            pltpu.async_copy(table.at[srcv.at[q]], rows.at[q], gsem.at[q])

        @pl.loop(0, CHUNKS_PW)
        def _(j):
            p = lax.rem(j, P)
            pn = lax.rem(j + PA, P)

            @pl.when(j >= PD)
            def _():
                pltpu.make_async_copy(
                    rows.at[pn], acc.at[pl.ds(0, CHUNK)], ssem.at[pn]
                ).wait()

            @pl.when(j + PA < CHUNKS_PW)
            def _():
                pltpu.async_copy(table.at[srcv.at[j + PA]], rows.at[pn], gsem.at[pn])

            pltpu.make_async_copy(
                table.at[pl.ds(0, CHUNK)], rows.at[p], gsem.at[p]
            ).wait()
            pltpu.async_copy(rows.at[p], acc.at[dstv.at[j]], ssem.at[p], add=True)

        for jt in range(CHUNKS_PW - PD, CHUNKS_PW):
            pltpu.make_async_copy(
                rows.at[jt % P], acc.at[pl.ds(0, CHUNK)], ssem.at[jt % P]
            ).wait()
        plsc.subcore_barrier()
        pltpu.sync_copy(
            acc.at[pl.ds(s * ROWS_PW, ROWS_PW)],
            out_hbm.at[h, c, pl.ds(s * ROWS_PW, ROWS_PW)],
        )


D_AGG = 64


def _sc_aggregate(tables, src_p, dst_p, zer):
    H = len(tables)
    return pl.kernel(
        _agg_body,
        out_type=jax.ShapeDtypeStruct((H, NC, N_PAD, D_AGG), jnp.float32),
        mesh=_mesh(),
        compiler_params=_SC_PARAMS,
        scratch_types=[
            pltpu.VMEM((CHUNKS_PW, CHUNK), jnp.int32),
            pltpu.VMEM((CHUNKS_PW, CHUNK), jnp.int32),
            pltpu.VMEM((P, CHUNK, D_AGG), jnp.float32),
            pltpu.VMEM_SHARED((N_PAD, D_AGG), jnp.float32),
            pltpu.VMEM_SHARED((N_PAD, D_AGG), jnp.float32),
            pltpu.SemaphoreType.DMA((P,)),
            pltpu.SemaphoreType.DMA((P,)),
        ],
    )(*tables, src_p, dst_p, zer)


_BR = 1024


def _mm_in_body(x_ref, wl_ref, wr_ref, d0_ref, d1_ref, ol_ref, or_ref):
    dinv = lax.rsqrt(d0_ref[:, :1] + d1_ref[:, :1] + 1.0)
    ol_ref[...] = (
        jnp.dot(x_ref[...], wl_ref[...], preferred_element_type=jnp.float32) * dinv
    )
    or_ref[...] = (
        jnp.dot(x_ref[...], wr_ref[...], preferred_element_type=jnp.float32) * dinv
    )


def _tc_in(x, wl, wr, d0, d1):
    k = x.shape[1]
    half = jax.ShapeDtypeStruct((N_PAD, D_AGG), jnp.float32)
    return pl.pallas_call(
        _mm_in_body,
        grid=(N_PAD // _BR,),
        in_specs=[
            pl.BlockSpec((_BR, k), lambda i: (i, 0)),
            pl.BlockSpec((k, D_AGG), lambda i: (0, 0)),
            pl.BlockSpec((k, D_AGG), lambda i: (0, 0)),
            pl.BlockSpec((_BR, DEG_W), lambda i: (i, 0)),
            pl.BlockSpec((_BR, DEG_W), lambda i: (i, 0)),
        ],
        out_specs=[
            pl.BlockSpec((_BR, D_AGG), lambda i: (i, 0)),
            pl.BlockSpec((_BR, D_AGG), lambda i: (i, 0)),
        ],
        out_shape=[half, half],
    )(x, wl, wr, d0, d1)


def _mid_body(a0l, a1l, gl, a0r, a1r, gr, d0, d1, bl, br, wt, wb, o_ref):
    dinv = lax.rsqrt(d0[:, :1] + d1[:, :1] + 1.0)
    hl = jnp.maximum(dinv * (a0l[...] + a1l[...] + gl[...]) + bl[...], 0.0)
    hr = jnp.maximum(dinv * (a0r[...] + a1r[...] + gr[...]) + br[...], 0.0)
    o_ref[...] = (
        jnp.dot(hl, wt[...], preferred_element_type=jnp.float32)
        + jnp.dot(hr, wb[...], preferred_element_type=jnp.float32)
    ) * dinv


def _tc_mid(a0l, a1l, gl, a0r, a1r, gr, d0, d1, bl, br, wt, wb, dout):
    half = pl.BlockSpec((_BR, D_AGG), lambda i: (i, 0))
    degs = pl.BlockSpec((_BR, DEG_W), lambda i: (i, 0))
    return pl.pallas_call(
        _mid_body,
        grid=(N_PAD // _BR,),
        in_specs=[
            half, half, half, half, half, half, degs, degs,
            pl.BlockSpec((1, D_AGG), lambda i: (0, 0)),
            pl.BlockSpec((1, D_AGG), lambda i: (0, 0)),
            pl.BlockSpec((D_AGG, dout), lambda i: (0, 0)),
            pl.BlockSpec((D_AGG, dout), lambda i: (0, 0)),
        ],
        out_specs=pl.BlockSpec((_BR, dout), lambda i: (i, 0)),
        out_shape=jax.ShapeDtypeStruct((N_PAD, dout), jnp.float32),
    )(a0l, a1l, gl, a0r, a1r, gr, d0, d1, bl, br, wt, wb)


def _out_body(a0_ref, a1_ref, g_ref, d0_ref, d1_ref, b_ref, w_ref, bc_ref, o_ref):
    dinv = lax.rsqrt(d0_ref[:, :1] + d1_ref[:, :1] + 1.0)
    h = jnp.maximum(
        dinv * (a0_ref[...] + a1_ref[...] + g_ref[...]) + b_ref[...], 0.0
    )
    o_ref[...] = (
        jnp.dot(h, w_ref[...], preferred_element_type=jnp.float32) + bc_ref[...]
    )


def _tc_out(a0, a1, g, d0, d1, b, w, bc, dout):
    k = g.shape[1]
    return pl.pallas_call(
        _out_body,
        grid=(N_PAD // _BR,),
        in_specs=[
            pl.BlockSpec((_BR, k), lambda i: (i, 0)),
            pl.BlockSpec((_BR, k), lambda i: (i, 0)),
            pl.BlockSpec((_BR, k), lambda i: (i, 0)),
            pl.BlockSpec((_BR, DEG_W), lambda i: (i, 0)),
            pl.BlockSpec((_BR, DEG_W), lambda i: (i, 0)),
            pl.BlockSpec((1, k), lambda i: (0, 0)),
            pl.BlockSpec((k, dout), lambda i: (0, 0)),
            pl.BlockSpec((1, dout), lambda i: (0, 0)),
        ],
        out_specs=pl.BlockSpec((_BR, dout), lambda i: (i, 0)),
        out_shape=jax.ShapeDtypeStruct((N_PAD, dout), jnp.float32),
    )(a0, a1, g, d0, d1, b, w, bc)


def kernel(X, edge_index, W1, b1, W2, b2, Wc, bc):
    f32 = jnp.float32
    src = edge_index[0].astype(jnp.int32)
    dst = edge_index[1].astype(jnp.int32)
    npad = E_PAD - N_EDGES
    pad_dst = N_NODES + (jnp.arange(npad, dtype=jnp.int32) % (N_PAD - N_NODES))
    src_p = jnp.concatenate([src, jnp.zeros((npad,), jnp.int32)]).reshape(
        E_PAD // CHUNK, CHUNK
    )
    dst_p = jnp.concatenate([dst, pad_dst]).reshape(E_PAD // CHUNK, CHUNK)

    Xp = jnp.concatenate([X, jnp.zeros((N_PAD - N_NODES, X.shape[1]), f32)])
    ones_c = jnp.ones((CHUNK, DEG_W), f32)
    zer_r1 = jnp.zeros((ROWS_PW, DEG_W), f32)
    zer64 = jnp.zeros((CHUNK, 64), f32)

    degs = _sc_degree(dst_p, ones_c, zer_r1)
    d0, d1 = degs[0], degs[1]

    g1l, g1r = _tc_in(Xp, W1[:, :D_AGG], W1[:, D_AGG:], d0, d1)
    a1 = _sc_aggregate([g1l, g1r], src_p, dst_p, zer64)
    g2 = _tc_mid(
        a1[0, 0], a1[0, 1], g1l, a1[1, 0], a1[1, 1], g1r, d0, d1,
        b1[:D_AGG].reshape(1, -1), b1[D_AGG:].reshape(1, -1),
        W2[:D_AGG], W2[D_AGG:], 64,
    )
    a2 = _sc_aggregate([g2], src_p, dst_p, zer64)
    logits = _tc_out(
        a2[0, 0], a2[0, 1], g2, d0, d1, b2.reshape(1, -1), Wc, bc.reshape(1, -1), 32
    )
    return logits[:N_NODES]
